# baseline (speedup 1.0000x reference)
import jax
import jax.numpy as jnp
from jax.experimental import pallas as pl

_B = 16384
_D = 128
_K = 38
_ROWS = 8192


def _body(feat_ref, w1_ref, b1_ref, w2_ref, b2_ref, w3_ref, b3_ref, out_ref):
    xt = feat_ref[:].T
    h = jnp.maximum(
        jnp.dot(w1_ref[:], xt, preferred_element_type=jnp.float32) + b1_ref[:], 0.0
    )
    h = jnp.dot(w2_ref[:], h, preferred_element_type=jnp.float32) + b2_ref[:]
    t = jnp.dot(w3_ref[:], h, preferred_element_type=jnp.float32) + b3_ref[:]
    x2 = 1.0 / (1.0 + jnp.exp(-t))

    xi = jax.lax.bitcast_convert_type(x2, jnp.int32)
    prefix = jnp.zeros((1, xt.shape[1]), dtype=jnp.int32)
    kk = jnp.full((1, xt.shape[1]), _K - 1, dtype=jnp.int32)
    for b in range(29, -1, -1):
        match = (xi ^ prefix) < jnp.int32(1 << b)
        c0 = jnp.sum(match.astype(jnp.int32), axis=0, keepdims=True)
        go1 = kk >= c0
        prefix = jnp.where(go1, prefix | jnp.int32(1 << b), prefix)
        kk = jnp.where(go1, kk - c0, kk)
    out_ref[:] = jnp.where(xi > prefix, x2, 0.0).T


@jax.jit
def kernel(feature, W1, b1, W2, b2, W3, b3):
    bb1 = b1.reshape(_D, 1)
    bb2 = b2.reshape(_D, 1)
    bb3 = b3.reshape(_D, 1)
    grid = _B // _ROWS
    row_spec = pl.BlockSpec((_ROWS, _D), lambda i: (i, 0))
    w_spec = pl.BlockSpec((_D, _D), lambda i: (0, 0))
    b_spec = pl.BlockSpec((_D, 1), lambda i: (0, 0))
    return pl.pallas_call(
        _body,
        grid=(grid,),
        in_specs=[row_spec, w_spec, b_spec, w_spec, b_spec, w_spec, b_spec],
        out_specs=row_spec,
        out_shape=jax.ShapeDtypeStruct((_B, _D), jnp.float32),
    )(feature, W1, bb1, W2, bb2, W3, bb3)


# speedup vs baseline: 1.1630x; 1.1630x over previous
import jax
import jax.numpy as jnp
from jax.experimental import pallas as pl

_B = 16384
_D = 128
_K = 38
_ROWS = 4096


def _body(feat_ref, w1_ref, b1_ref, w2_ref, b2_ref, w3_ref, b3_ref, out_ref):
    xt = feat_ref[:].T
    h = jnp.maximum(
        jnp.dot(w1_ref[:], xt, preferred_element_type=jnp.float32) + b1_ref[:], 0.0
    )
    h = jnp.dot(w2_ref[:], h, preferred_element_type=jnp.float32) + b2_ref[:]
    t = jnp.dot(w3_ref[:], h, preferred_element_type=jnp.float32) + b3_ref[:]
    x2 = 1.0 / (1.0 + jnp.exp(-t))

    xi = jax.lax.bitcast_convert_type(x2, jnp.int32)
    lo = jnp.zeros((1, xt.shape[1]), dtype=jnp.int32)
    hi = jnp.full((1, xt.shape[1]), jnp.int32(0x3F800000))
    for _ in range(30):
        mid = jax.lax.shift_right_logical(lo + hi, 1)
        c = jnp.sum((xi <= mid).astype(jnp.int32), axis=0, keepdims=True)
        geq = c >= _K
        hi = jnp.where(geq, mid, hi)
        lo = jnp.where(geq, lo, mid + 1)
    out_ref[:] = jnp.where(xi > lo, x2, 0.0).T


@jax.jit
def kernel(feature, W1, b1, W2, b2, W3, b3):
    bb1 = b1.reshape(_D, 1)
    bb2 = b2.reshape(_D, 1)
    bb3 = b3.reshape(_D, 1)
    grid = _B // _ROWS
    row_spec = pl.BlockSpec((_ROWS, _D), lambda i: (i, 0))
    w_spec = pl.BlockSpec((_D, _D), lambda i: (0, 0))
    b_spec = pl.BlockSpec((_D, 1), lambda i: (0, 0))
    return pl.pallas_call(
        _body,
        grid=(grid,),
        in_specs=[row_spec, w_spec, b_spec, w_spec, b_spec, w_spec, b_spec],
        out_specs=row_spec,
        out_shape=jax.ShapeDtypeStruct((_B, _D), jnp.float32),
    )(feature, W1, bb1, W2, bb2, W3, bb3)
